# baseline (speedup 1.0000x reference)
import functools

import jax
import jax.numpy as jnp
from jax import lax
from jax.experimental import pallas as pl
from jax.experimental.pallas import tpu as pltpu
from jax.experimental.pallas import tpu_sc as plsc

U_SIZE = 1000000
I_SIZE = 100000
DIM = 64
BATCH = 16384

NUM_CORES = 2
NUM_SUBCORES = 16
NUM_WORKERS = NUM_CORES * NUM_SUBCORES
ROWS_PER_WORKER = BATCH // NUM_WORKERS
GATHER_CHUNK = 128
NUM_CHUNKS = ROWS_PER_WORKER // GATHER_CHUNK
GROUPS = ROWS_PER_WORKER // 16


def _make_kernel():
  mesh = plsc.VectorSubcoreMesh(core_axis_name="c", subcore_axis_name="s")

  @functools.partial(
      pl.kernel,
      mesh=mesh,
      out_type=jax.ShapeDtypeStruct((BATCH,), jnp.float32),
      compiler_params=pltpu.CompilerParams(
          needs_layout_passes=False, use_tc_tiling_on_sc=False),
      scratch_types=[
          pltpu.VMEM((ROWS_PER_WORKER,), jnp.int32),
          pltpu.VMEM((ROWS_PER_WORKER,), jnp.int32),
          pltpu.VMEM((ROWS_PER_WORKER, DIM), jnp.float32),
          pltpu.VMEM((ROWS_PER_WORKER, DIM), jnp.float32),
          pltpu.VMEM((ROWS_PER_WORKER,), jnp.float32),
          pltpu.SemaphoreType.DMA,
      ],
  )
  def k(users_hbm, items_hbm, user_emb_hbm, item_emb_hbm, out_hbm,
        uidx_v, iidx_v, urows_v, irows_v, out_v, sem):
    wid = lax.axis_index("s") * NUM_CORES + lax.axis_index("c")
    base = wid * ROWS_PER_WORKER

    pltpu.sync_copy(users_hbm.at[pl.ds(base, ROWS_PER_WORKER)], uidx_v)
    pltpu.sync_copy(items_hbm.at[pl.ds(base, ROWS_PER_WORKER)], iidx_v)

    copies = []
    for j in range(NUM_CHUNKS):
      off = j * GATHER_CHUNK
      copies.append(pltpu.async_copy(
          user_emb_hbm.at[uidx_v.at[pl.ds(off, GATHER_CHUNK)]],
          urows_v.at[pl.ds(off, GATHER_CHUNK)], sem))
      copies.append(pltpu.async_copy(
          item_emb_hbm.at[iidx_v.at[pl.ds(off, GATHER_CHUNK)]],
          irows_v.at[pl.ds(off, GATHER_CHUNK)], sem))
    for c in copies:
      c.wait()

    lanes = lax.iota(jnp.int32, 16)

    def group_body(g, _):
      row_idx = g * 16 + lanes
      acc = jnp.zeros((16,), jnp.float32)
      for d in range(DIM):
        col = (lanes + d) & (DIM - 1)
        ug = plsc.load_gather(urows_v, [row_idx, col])
        ig = plsc.load_gather(irows_v, [row_idx, col])
        acc = acc + ug * ig
      out_v[pl.ds(g * 16, 16)] = acc
      return _

    lax.fori_loop(0, GROUPS, group_body, 0, unroll=False)

    pltpu.sync_copy(out_v, out_hbm.at[pl.ds(base, ROWS_PER_WORKER)])

  return k


_kernel_call = _make_kernel()


@jax.jit
def kernel(users, items, user_emb, item_emb):
  return _kernel_call(users, items, user_emb, item_emb)
